# baseline (speedup 1.0000x reference)
import jax
import jax.numpy as jnp
from jax.experimental import pallas as pl
from jax.experimental.pallas import tpu as pltpu

_DEPTH = 3


def _make_ms_cam_kernel(N):
    def _ms_cam_kernel(x_ref, w1_ref, b1_ref, w2_ref, b2_ref,
                       g1_ref, gb1_ref, g2_ref, gb2_ref, o_ref,
                       xbuf, obuf, w1r, w2r, g1r, g2r,
                       w1s, b1s, w2s, bss, g1s, gb1s, g2s, gb2s,
                       wsems, isems, osems):
        copies = [(w1_ref, w1r), (w2_ref, w2r), (g1_ref, g1r),
                  (g2_ref, g2r), (b1_ref, b1s), (gb1_ref, gb1s),
                  (b2_ref, bss), (gb2_ref, gb2s)]
        for i, (src, dst) in enumerate(copies):
            pltpu.make_async_copy(src, dst, wsems.at[i]).start()

        for k in range(_DEPTH):
            pltpu.make_async_copy(x_ref.at[k], xbuf.at[k], isems.at[k]).start()

        for i, (src, dst) in enumerate(copies):
            pltpu.make_async_copy(src, dst, wsems.at[i]).wait()
        w1s[...] = w1r[...].T
        w2s[...] = w2r[...].T
        g1s[...] = g1r[...].T
        g2s[...] = g2r[...].T
        bss[...] = bss[...] + gb2s[...]

        for n in range(N):
            s = n % _DEPTH
            pltpu.make_async_copy(x_ref.at[n], xbuf.at[s], isems.at[s]).wait()
            x = xbuf[s]
            C, HW = x.shape

            m = jnp.sum(x, axis=1, keepdims=True) * (1.0 / HW)
            mb = jnp.broadcast_to(m, (C, 128))
            hg = jnp.maximum(
                jnp.dot(g1s[...], mb, preferred_element_type=jnp.float32)
                + gb1s[...], 0.0)
            xg = jnp.dot(g2s[...], hg,
                         preferred_element_type=jnp.float32)[:, 0:1]

            h = jnp.maximum(
                jnp.dot(w1s[...], x, preferred_element_type=jnp.float32)
                + b1s[...], 0.0)
            xl = jnp.dot(w2s[...], h, preferred_element_type=jnp.float32)

            if n >= _DEPTH:
                pltpu.make_async_copy(obuf.at[s], o_ref.at[n - _DEPTH],
                                      osems.at[s]).wait()

            gate = 0.5 + 0.5 * jnp.tanh((xl + (xg + bss[...])) * 0.5)
            obuf[s] = (x * gate).astype(obuf.dtype)

            pltpu.make_async_copy(obuf.at[s], o_ref.at[n], osems.at[s]).start()
            if n + _DEPTH < N:
                pltpu.make_async_copy(x_ref.at[n + _DEPTH],
                                      xbuf.at[(n + _DEPTH) % _DEPTH],
                                      isems.at[(n + _DEPTH) % _DEPTH]).start()

        for n in range(max(N - _DEPTH, 0), N):
            s = n % _DEPTH
            pltpu.make_async_copy(obuf.at[s], o_ref.at[n], osems.at[s]).wait()
    return _ms_cam_kernel


def kernel(x_nchw, w1, b1, w2, b2, g1, gb1, g2, gb2):
    N, C, H, W = x_nchw.shape
    HW = H * W
    Ci = w1.shape[1]

    x = x_nchw.reshape(N, C, HW)
    b1c = b1.reshape(Ci, 1)
    b2c = b2.reshape(C, 1)
    gb1c = gb1.reshape(Ci, 1)
    gb2c = gb2.reshape(C, 1)

    anyspec = pl.BlockSpec(memory_space=pl.ANY)
    out = pl.pallas_call(
        _make_ms_cam_kernel(N),
        out_shape=jax.ShapeDtypeStruct((N, C, HW), x.dtype),
        in_specs=[anyspec] * 9,
        out_specs=anyspec,
        scratch_shapes=[
            pltpu.VMEM((_DEPTH, C, HW), jnp.float32),
            pltpu.VMEM((_DEPTH, C, HW), jnp.float32),
            pltpu.VMEM((C, Ci), jnp.float32),
            pltpu.VMEM((Ci, C), jnp.float32),
            pltpu.VMEM((C, Ci), jnp.float32),
            pltpu.VMEM((Ci, C), jnp.float32),
            pltpu.VMEM((Ci, C), jnp.float32),
            pltpu.VMEM((Ci, 1), jnp.float32),
            pltpu.VMEM((C, Ci), jnp.float32),
            pltpu.VMEM((C, 1), jnp.float32),
            pltpu.VMEM((Ci, C), jnp.float32),
            pltpu.VMEM((Ci, 1), jnp.float32),
            pltpu.VMEM((C, Ci), jnp.float32),
            pltpu.VMEM((C, 1), jnp.float32),
            pltpu.SemaphoreType.DMA((8,)),
            pltpu.SemaphoreType.DMA((_DEPTH,)),
            pltpu.SemaphoreType.DMA((_DEPTH,)),
        ],
    )(x, w1, b1c, w2, b2c, g1, gb1c, g2, gb2c)

    return out.reshape(N, C, H, W)


# speedup vs baseline: 1.0302x; 1.0302x over previous
import jax
import jax.numpy as jnp
from jax.experimental import pallas as pl
from jax.experimental.pallas import tpu as pltpu


def _make_ms_cam_kernel(Ci, NB):
    def _ms_cam_kernel(x_ref, w_ref, b_ref, o_ref):
        r0 = 2 * Ci
        for i in range(NB):
            x = x_ref[i]
            C, HW = x.shape

            m = jnp.sum(x, axis=1, keepdims=True) * (1.0 / HW)
            mb = jnp.broadcast_to(m, (C, 128))
            hg = jnp.maximum(
                jnp.dot(w_ref[Ci:r0, :], mb, preferred_element_type=jnp.float32)
                + b_ref[Ci:r0, :], 0.0)
            xg = jnp.dot(w_ref[r0:, Ci:r0], hg,
                         preferred_element_type=jnp.float32)[:, 0:1]

            h = jnp.maximum(
                jnp.dot(w_ref[0:Ci, :], x, preferred_element_type=jnp.float32)
                + b_ref[0:Ci, :], 0.0)
            xl = jnp.dot(w_ref[r0:, 0:Ci], h,
                         preferred_element_type=jnp.float32)

            z = xl + (xg + b_ref[r0:, :])
            gate = 0.5 + 0.5 * jnp.tanh(z * 0.5)
            o_ref[i] = (x * gate).astype(o_ref.dtype)
    return _ms_cam_kernel


def kernel(x_nchw, w1, b1, w2, b2, g1, gb1, g2, gb2):
    N, C, H, W = x_nchw.shape
    HW = H * W
    Ci = w1.shape[1]

    x = x_nchw.reshape(N, C, HW)

    bot = jnp.concatenate([w2.T, g2.T], axis=1)
    if 2 * Ci < C:
        bot = jnp.pad(bot, ((0, 0), (0, C - 2 * Ci)))
    wpack = jnp.concatenate([w1.T, g1.T, bot], axis=0)
    bpack = jnp.concatenate([b1, gb1, b2 + gb2]).reshape(2 * Ci + C, 1)

    NB = 2 if N % 2 == 0 else 1
    const = lambda shape: pl.BlockSpec(shape, lambda n: (0,) * len(shape))
    out = pl.pallas_call(
        _make_ms_cam_kernel(Ci, NB),
        out_shape=jax.ShapeDtypeStruct((N, C, HW), x.dtype),
        grid=(N // NB,),
        in_specs=[
            pl.BlockSpec((NB, C, HW), lambda n: (n, 0, 0)),
            const((2 * Ci + C, C)),
            const((2 * Ci + C, 1)),
        ],
        out_specs=pl.BlockSpec((NB, C, HW), lambda n: (n, 0, 0)),
        compiler_params=pltpu.CompilerParams(
            dimension_semantics=("parallel",)),
    )(x, wpack, bpack)

    return out.reshape(N, C, H, W)
